# baseline (speedup 1.0000x reference)
import jax
import jax.numpy as jnp
from jax import lax
from jax.experimental import pallas as pl
from jax.experimental.pallas import tpu as pltpu
from jax.experimental.pallas import tpu_sc as plsc

N = 1025
H = 16
HG = 4
RPC = 8
NW = 32
CHUNKS_PER_W = 4
UNROLL = 8


def _sc_body(idx_hbm, w_hbm, out_hbm, idx_v, bufs, wlut_v, sems):
    cid = lax.axis_index("c")
    sid = lax.axis_index("s")
    wid = sid * 2 + cid

    pltpu.sync_copy(w_hbm, wlut_v)

    lanes = lax.iota(jnp.int32, 16)
    tail_cols = lanes + (N - 16)

    def do_rows(r0, nrows):
        pltpu.sync_copy(
            idx_hbm.at[pl.ds(r0, nrows), :],
            idx_v.at[pl.ds(0, nrows), :],
        )

        def compute_group(g):
            grp = bufs[(g % 2) * HG:(g % 2) * HG + HG]

            @plsc.parallel_loop(0, nrows * (N // 16), unroll=UNROLL)
            def vec_body(i):
                r = i // (N // 16)
                off = pl.multiple_of((i % (N // 16)) * 16, 16)
                vec = idx_v[r, pl.ds(off, 16)]
                for k in range(HG):
                    grp[k][r, pl.ds(off, 16)] = plsc.load_gather(
                        wlut_v, [vec + (g * HG + k) * 64]
                    )
            for r in range(nrows):
                rows16 = jnp.full((16,), r, jnp.int32)
                vec = plsc.load_gather(idx_v, [rows16, tail_cols])
                for k in range(HG):
                    vals = plsc.load_gather(wlut_v, [vec + (g * HG + k) * 64])
                    plsc.store_scatter(grp[k], [rows16, tail_cols], vals)

        def fire_group(g):
            sem = sems[g % 2]
            for k in range(HG):
                pltpu.async_copy(
                    bufs[(g % 2) * HG + k].at[pl.ds(0, nrows), :],
                    out_hbm.at[g * HG + k, pl.ds(r0, nrows), :],
                    sem,
                )

        def drain_group(parity):
            sem = sems[parity]
            for k in range(HG):
                pltpu.make_async_copy(
                    bufs[parity * HG + k].at[pl.ds(0, nrows), :],
                    out_hbm.at[0, pl.ds(0, nrows), :],
                    sem,
                ).wait()

        for g in range(H // HG):
            if g >= 2:
                drain_group(g % 2)
            compute_group(g)
            fire_group(g)
        drain_group(0)
        drain_group(1)

    def chunk_body(c, carry):
        do_rows((wid * CHUNKS_PER_W + c) * RPC, RPC)
        return carry

    lax.fori_loop(0, CHUNKS_PER_W, chunk_body, 0)

    @pl.when(wid == NW - 1)
    def _():
        do_rows(N - 1, 1)


def _body(idx_hbm, w_hbm, out_hbm, idx_v,
          b0, b1, b2, b3, b4, b5, b6, b7, wlut_v, sem0, sem1):
    _sc_body(idx_hbm, w_hbm, out_hbm, idx_v,
             [b0, b1, b2, b3, b4, b5, b6, b7], wlut_v, [sem0, sem1])


def kernel(spatial_bias, weight):
    wflat = weight.T.reshape(-1)
    mesh = plsc.VectorSubcoreMesh(core_axis_name="c", subcore_axis_name="s")
    run = pl.kernel(
        _body,
        mesh=mesh,
        compiler_params=pltpu.CompilerParams(needs_layout_passes=False),
        out_type=jax.ShapeDtypeStruct((H, N, N), jnp.float32),
        scratch_types=(
            [pltpu.VMEM((RPC, N), jnp.int32)]
            + [pltpu.VMEM((RPC, N), jnp.float32)] * 8
            + [pltpu.VMEM((2 * 32 * H,), jnp.float32)]
            + [pltpu.SemaphoreType.DMA] * 2
        ),
    )
    out = run(spatial_bias, wflat)
    return out.reshape(1, H, N, N)


# speedup vs baseline: 1.0052x; 1.0052x over previous
import jax
import jax.numpy as jnp
from jax import lax
from jax.experimental import pallas as pl
from jax.experimental.pallas import tpu as pltpu
from jax.experimental.pallas import tpu_sc as plsc

N = 1025
H = 16
HG = 4
RPC = 8
NW = 32
CHUNKS_PER_W = 4
UNROLL = 4


def _sc_body(idx_hbm, w_hbm, out_hbm, idx_v, bufs, wlut_v, sems):
    cid = lax.axis_index("c")
    sid = lax.axis_index("s")
    wid = sid * 2 + cid

    pltpu.sync_copy(w_hbm, wlut_v)

    lanes = lax.iota(jnp.int32, 16)
    tail_cols = lanes + (N - 16)

    def do_rows(r0, nrows):
        pltpu.sync_copy(
            idx_hbm.at[pl.ds(r0, nrows), :],
            idx_v.at[pl.ds(0, nrows), :],
        )

        def compute_group(g):
            grp = bufs[(g % 2) * HG:(g % 2) * HG + HG]

            @plsc.parallel_loop(0, nrows * (N // 16), unroll=UNROLL)
            def vec_body(i):
                r = i // (N // 16)
                off = pl.multiple_of((i % (N // 16)) * 16, 16)
                vec = idx_v[r, pl.ds(off, 16)]
                for k in range(HG):
                    grp[k][r, pl.ds(off, 16)] = plsc.load_gather(
                        wlut_v, [vec + (g * HG + k) * 64]
                    )
            for r in range(nrows):
                rows16 = jnp.full((16,), r, jnp.int32)
                vec = plsc.load_gather(idx_v, [rows16, tail_cols])
                for k in range(HG):
                    vals = plsc.load_gather(wlut_v, [vec + (g * HG + k) * 64])
                    plsc.store_scatter(grp[k], [rows16, tail_cols], vals)

        def fire_group(g):
            sem = sems[g % 2]
            for k in range(HG):
                pltpu.async_copy(
                    bufs[(g % 2) * HG + k].at[pl.ds(0, nrows), :],
                    out_hbm.at[g * HG + k, pl.ds(r0, nrows), :],
                    sem,
                )

        def drain_group(parity):
            sem = sems[parity]
            for k in range(HG):
                pltpu.make_async_copy(
                    bufs[parity * HG + k].at[pl.ds(0, nrows), :],
                    out_hbm.at[0, pl.ds(0, nrows), :],
                    sem,
                ).wait()

        for g in range(H // HG):
            if g >= 2:
                drain_group(g % 2)
            compute_group(g)
            fire_group(g)
        drain_group(0)
        drain_group(1)

    def chunk_body(c, carry):
        do_rows((wid * CHUNKS_PER_W + c) * RPC, RPC)
        return carry

    lax.fori_loop(0, CHUNKS_PER_W, chunk_body, 0)

    @pl.when(wid == NW - 1)
    def _():
        do_rows(N - 1, 1)


def _body(idx_hbm, w_hbm, out_hbm, idx_v,
          b0, b1, b2, b3, b4, b5, b6, b7, wlut_v, sem0, sem1):
    _sc_body(idx_hbm, w_hbm, out_hbm, idx_v,
             [b0, b1, b2, b3, b4, b5, b6, b7], wlut_v, [sem0, sem1])


def kernel(spatial_bias, weight):
    wflat = weight.T.reshape(-1)
    mesh = plsc.VectorSubcoreMesh(core_axis_name="c", subcore_axis_name="s")
    run = pl.kernel(
        _body,
        mesh=mesh,
        compiler_params=pltpu.CompilerParams(needs_layout_passes=False),
        out_type=jax.ShapeDtypeStruct((H, N, N), jnp.float32),
        scratch_types=(
            [pltpu.VMEM((RPC, N), jnp.int32)]
            + [pltpu.VMEM((RPC, N), jnp.float32)] * 8
            + [pltpu.VMEM((2 * 32 * H,), jnp.float32)]
            + [pltpu.SemaphoreType.DMA] * 2
        ),
    )
    out = run(spatial_bias, wflat)
    return out.reshape(1, H, N, N)


# speedup vs baseline: 1.0674x; 1.0618x over previous
import jax
import jax.numpy as jnp
from jax import lax
from jax.experimental import pallas as pl
from jax.experimental.pallas import tpu as pltpu
from jax.experimental.pallas import tpu_sc as plsc

N = 1025
H = 16
HG = 4
RPC = 8
NW = 32
CHUNKS_PER_W = 4
UNROLL = 4


def _sc_body(idx_hbm, w_hbm, out_hbm, idx_v, bufs, wlut_v, sems):
    cid = lax.axis_index("c")
    sid = lax.axis_index("s")
    wid = sid * 2 + cid

    pltpu.sync_copy(w_hbm, wlut_v)

    lanes = lax.iota(jnp.int32, 16)
    tail_cols = lanes + (N - 16)

    def do_rows(r0, nrows):
        pltpu.sync_copy(
            idx_hbm.at[pl.ds(r0, nrows), :],
            idx_v.at[pl.ds(0, nrows), :],
        )

        def compute_group(g):
            grp = bufs[(g % 2) * HG:(g % 2) * HG + HG]

            @plsc.parallel_loop(0, nrows * (N // 16), unroll=UNROLL)
            def vec_body(i):
                r = i // (N // 16)
                off = pl.multiple_of((i % (N // 16)) * 16, 16)
                vec = idx_v[r, pl.ds(off, 16)]
                for k in range(HG):
                    grp[k][r, pl.ds(off, 16)] = plsc.load_gather(
                        wlut_v, [vec + (g * HG + k) * 64]
                    )
            for r in range(nrows):
                rows16 = jnp.full((16,), r, jnp.int32)
                vec = plsc.load_gather(idx_v, [rows16, tail_cols])
                for k in range(HG):
                    vals = plsc.load_gather(wlut_v, [vec + (g * HG + k) * 64])
                    plsc.store_scatter(grp[k], [rows16, tail_cols], vals)

        def fire_group(g):
            sem = sems[g % 2]
            for k in range(HG):
                pltpu.async_copy(
                    bufs[(g % 2) * HG + k].at[pl.ds(0, nrows), :],
                    out_hbm.at[0, g * HG + k, pl.ds(r0, nrows), :],
                    sem,
                )

        def drain_group(parity):
            sem = sems[parity]
            for k in range(HG):
                pltpu.make_async_copy(
                    bufs[parity * HG + k].at[pl.ds(0, nrows), :],
                    out_hbm.at[0, 0, pl.ds(0, nrows), :],
                    sem,
                ).wait()

        for g in range(H // HG):
            if g >= 2:
                drain_group(g % 2)
            compute_group(g)
            fire_group(g)
        drain_group(0)
        drain_group(1)

    def chunk_body(c, carry):
        do_rows((wid * CHUNKS_PER_W + c) * RPC, RPC)
        return carry

    lax.fori_loop(0, CHUNKS_PER_W, chunk_body, 0)

    @pl.when(wid == NW - 1)
    def _():
        do_rows(N - 1, 1)


def _body(idx_hbm, w_hbm, out_hbm, idx_v,
          b0, b1, b2, b3, b4, b5, b6, b7, wlut_v, sem0, sem1):
    _sc_body(idx_hbm, w_hbm, out_hbm, idx_v,
             [b0, b1, b2, b3, b4, b5, b6, b7], wlut_v, [sem0, sem1])


def kernel(spatial_bias, weight):
    wflat = weight.T.reshape(-1)
    mesh = plsc.VectorSubcoreMesh(core_axis_name="c", subcore_axis_name="s")
    run = pl.kernel(
        _body,
        mesh=mesh,
        compiler_params=pltpu.CompilerParams(needs_layout_passes=False),
        out_type=jax.ShapeDtypeStruct((1, H, N, N), jnp.float32),
        scratch_types=(
            [pltpu.VMEM((RPC, N), jnp.int32)]
            + [pltpu.VMEM((RPC, N), jnp.float32)] * 8
            + [pltpu.VMEM((2 * 32 * H,), jnp.float32)]
            + [pltpu.SemaphoreType.DMA] * 2
        ),
    )
    return run(spatial_bias, wflat)


# speedup vs baseline: 2.1134x; 1.9800x over previous
import jax
import jax.numpy as jnp
from jax import lax
from jax.experimental import pallas as pl
from jax.experimental.pallas import tpu as pltpu
from jax.experimental.pallas import tpu_sc as plsc

N = 1025
H = 16
RPC = 2
NW = 32
CHUNKS_PER_W = 16
UNROLL = 4


def _sc_body(idx_hbm, w_hbm, out_hbm, idx_v8, buf_a, buf_b,
             wlut_v, sem_a, sem_b):
    cid = lax.axis_index("c")
    sid = lax.axis_index("s")
    wid = sid * 2 + cid

    pltpu.sync_copy(w_hbm, wlut_v)

    lanes = lax.iota(jnp.int32, 16)
    tail_cols = lanes + (N - 16)

    def do_rows(r0, rq, nrows, idx_v, buf, sem, first):
        @pl.when(jnp.logical_not(first))
        def _():
            pltpu.make_async_copy(
                buf.at[pl.ds(0, nrows), :, :],
                out_hbm.at[0, pl.ds(0, nrows), :, :],
                sem,
            ).wait()

        @plsc.parallel_loop(0, nrows * (N // 16), unroll=UNROLL)
        def vec_body(i):
            r = i // (N // 16)
            off = pl.multiple_of((i % (N // 16)) * 16, 16)
            vec = idx_v[rq + r, pl.ds(off, 16)]
            for k in range(H):
                buf[r, k, pl.ds(off, 16)] = plsc.load_gather(
                    wlut_v, [vec + k * 64]
                )
        for r in range(nrows):
            rows16 = jnp.full((16,), rq + r, jnp.int32)
            vec = plsc.load_gather(idx_v, [rows16, tail_cols])
            for k in range(H):
                vals = plsc.load_gather(wlut_v, [vec + k * 64])
                plsc.store_scatter(
                    buf,
                    [jnp.full((16,), r, jnp.int32),
                     jnp.full((16,), k, jnp.int32),
                     tail_cols],
                    vals,
                )

        pltpu.async_copy(
            buf.at[pl.ds(0, nrows), :, :],
            out_hbm.at[0, pl.ds(r0, nrows), :, :],
            sem,
        )

    def final_drain(buf, sem, nrows):
        pltpu.make_async_copy(
            buf.at[pl.ds(0, nrows), :, :],
            out_hbm.at[0, pl.ds(0, nrows), :, :],
            sem,
        ).wait()

    base = wid * CHUNKS_PER_W * RPC

    def octet_body(o, carry):
        r0 = base + o * 8
        pltpu.sync_copy(idx_hbm.at[pl.ds(r0, 8), :], idx_v8)
        for q in range(4):
            buf, sem = (buf_a, sem_a) if q % 2 == 0 else (buf_b, sem_b)
            first = jnp.logical_and(o == 0, q < 2)
            do_rows(r0 + q * RPC, q * RPC, RPC, idx_v8, buf, sem, first)
        return carry

    lax.fori_loop(0, CHUNKS_PER_W * RPC // 8, octet_body, 0)
    final_drain(buf_a, sem_a, RPC)
    final_drain(buf_b, sem_b, RPC)

    @pl.when(wid == NW - 1)
    def _():
        pltpu.sync_copy(
            idx_hbm.at[pl.ds(N - 1, 1), :], idx_v8.at[pl.ds(0, 1), :]
        )
        do_rows(N - 1, 0, 1, idx_v8, buf_a, sem_a, jnp.bool_(True))
        final_drain(buf_a, sem_a, 1)


def kernel(spatial_bias, weight):
    wflat = weight.T.reshape(-1)
    mesh = plsc.VectorSubcoreMesh(core_axis_name="c", subcore_axis_name="s")
    run = pl.kernel(
        _sc_body,
        mesh=mesh,
        compiler_params=pltpu.CompilerParams(needs_layout_passes=False),
        out_type=jax.ShapeDtypeStruct((1, N, H, N), jnp.float32),
        scratch_types=[
            pltpu.VMEM((8, N), jnp.int32),
            pltpu.VMEM((RPC, H, N), jnp.float32),
            pltpu.VMEM((RPC, H, N), jnp.float32),
            pltpu.VMEM((2 * 32 * H,), jnp.float32),
            pltpu.SemaphoreType.DMA,
            pltpu.SemaphoreType.DMA,
        ],
    )
    out = run(spatial_bias, wflat)
    return jnp.transpose(out, (0, 2, 1, 3))
